# baseline (speedup 1.0000x reference)
import functools

import jax
import jax.numpy as jnp
from jax import lax
from jax.experimental import pallas as pl
from jax.experimental.pallas import tpu as pltpu
from jax.experimental.pallas import tpu_sc as plsc

VOCAB = 100000
EMBED = 64
SEQ = 50
ROWS = 4096

NC = 2
NS = 16
NW = NC * NS

R_PER_W = ROWS // NW
G_ROWS = 8
N_GROUPS = R_PER_W // G_ROWS
NBUF = 2
K = N_GROUPS // NBUF

_mesh = plsc.VectorSubcoreMesh(core_axis_name="c", subcore_axis_name="s")


@functools.partial(
    pl.kernel,
    mesh=_mesh,
    out_type=jax.ShapeDtypeStruct((ROWS, SEQ, EMBED), jnp.float32),
    compiler_params=pltpu.CompilerParams(use_tc_tiling_on_sc=False),
    scratch_types=[
        pltpu.VMEM((R_PER_W, SEQ), jnp.int32),
        pltpu.VMEM((NBUF, G_ROWS, SEQ, EMBED), jnp.float32),
        *([pltpu.SemaphoreType.DMA] * NBUF),
        *([pltpu.SemaphoreType.DMA] * NBUF),
    ],
)
def _emb_lookup(idx_hbm, table_hbm, out_hbm, idx_v, rows_v, *sems):
    gsem = sems[:NBUF]
    ssem = sems[NBUF:]
    wid = lax.axis_index("s") * NC + lax.axis_index("c")
    base = wid * R_PER_W
    pltpu.sync_copy(idx_hbm.at[pl.ds(base, R_PER_W)], idx_v)

    def gather_start(g, b):
        for j in range(G_ROWS):
            pltpu.async_copy(
                table_hbm.at[idx_v.at[g * G_ROWS + j]],
                rows_v.at[b, j], gsem[b])

    def gather_wait(g, b):
        for j in range(G_ROWS):
            pltpu.make_async_copy(
                table_hbm.at[idx_v.at[g * G_ROWS + j]],
                rows_v.at[b, j], gsem[b]).wait()

    def scatter_start(g, b):
        off = pl.multiple_of(base + g * G_ROWS, G_ROWS)
        pltpu.async_copy(rows_v.at[b], out_hbm.at[pl.ds(off, G_ROWS)],
                         ssem[b])

    def scatter_wait(g, b):
        off = pl.multiple_of(base + g * G_ROWS, G_ROWS)
        pltpu.make_async_copy(rows_v.at[b], out_hbm.at[pl.ds(off, G_ROWS)],
                              ssem[b]).wait()

    for b in range(NBUF):
        gather_start(b, b)

    def body(i, carry):
        g0 = i * NBUF
        for b in range(NBUF):
            g = g0 + b
            gather_wait(g, b)
            scatter_start(g, b)
        for b in range(NBUF):
            g = g0 + b
            scatter_wait(g, b)
            gather_start(g + NBUF, b)
        return carry

    lax.fori_loop(0, K - 1, body, 0)

    g0 = (K - 1) * NBUF
    for b in range(NBUF):
        g = g0 + b
        gather_wait(g, b)
        scatter_start(g, b)
    for b in range(NBUF):
        scatter_wait(g0 + b, b)


def kernel(x, table):
    return _emb_lookup(x.astype(jnp.int32), table)
